# baseline (speedup 1.0000x reference)
import functools

import jax
import jax.numpy as jnp
from jax import lax
from jax.experimental import pallas as pl
from jax.experimental.pallas import tpu as pltpu
from jax.experimental.pallas import tpu_sc as plsc

N_ROWS = 100000
D = 64
B_ROWS = 16384

NC = 2
NS = 16
L = 16

CHUNKS_PER_CORE = 4
CHUNK = 12504
LAST_CHUNK = N_ROWS - (NC * CHUNKS_PER_CORE - 1) * CHUNK
TRASH = 64

B_SUB = 128
N_SUB = B_ROWS // (B_SUB * NS)

A_PER_TILE = 784
A_LAST = CHUNK - (NS - 1) * A_PER_TILE
A_LAST_FINAL = LAST_CHUNK - (NS - 1) * A_PER_TILE

_mesh = plsc.VectorSubcoreMesh(core_axis_name="c", subcore_axis_name="s")


@functools.partial(
    pl.kernel,
    mesh=_mesh,
    out_type=jax.ShapeDtypeStruct((N_ROWS, D), jnp.float32),
    scratch_types=[
        pltpu.VMEM((B_SUB, D), jnp.float32),
        pltpu.VMEM((B_SUB,), jnp.int32),
        pltpu.VMEM((1, B_SUB), jnp.int32),
        pltpu.VMEM_SHARED((CHUNK + TRASH, D), jnp.float32),
    ],
    compiler_params=pltpu.CompilerParams(use_tc_tiling_on_sc=True),
)
def _scatter_add(idx_hbm, a_hbm, b_hbm, out_hbm, b_v, idx_v, sidx_v, chunk_sh):
    c = lax.axis_index("c")
    s = lax.axis_index("s")
    lanes = lax.iota(jnp.int32, L)

    for k in range(CHUNKS_PER_CORE):
        base = (c * CHUNKS_PER_CORE + k) * CHUNK
        is_final = k == CHUNKS_PER_CORE - 1

        @pl.when(s < NS - 1)
        def _():
            pltpu.sync_copy(
                a_hbm.at[pl.ds(base + s * A_PER_TILE, A_PER_TILE)],
                chunk_sh.at[pl.ds(s * A_PER_TILE, A_PER_TILE)],
            )

        @pl.when((s == NS - 1) & (jnp.bool_(not is_final) | (c == 0)))
        def _():
            pltpu.sync_copy(
                a_hbm.at[pl.ds(base + (NS - 1) * A_PER_TILE, A_LAST)],
                chunk_sh.at[pl.ds((NS - 1) * A_PER_TILE, A_LAST)],
            )

        if is_final:
            @pl.when((s == NS - 1) & (c == 1))
            def _():
                pltpu.sync_copy(
                    a_hbm.at[pl.ds(base + (NS - 1) * A_PER_TILE, A_LAST_FINAL)],
                    chunk_sh.at[pl.ds((NS - 1) * A_PER_TILE, A_LAST_FINAL)],
                )

        plsc.subcore_barrier()

        for sub in range(N_SUB):
            off = sub * (B_SUB * NS) + s * B_SUB
            pltpu.sync_copy(b_hbm.at[pl.ds(off, B_SUB)], b_v)
            pltpu.sync_copy(idx_hbm.at[pl.ds(off, B_SUB)], idx_v)
            for g in range(B_SUB // L):
                v = idx_v[pl.ds(g * L, L)]
                local = v - base
                in_chunk = (local >= 0) & (local < CHUNK)
                trash = lanes + jnp.int32(CHUNK + (g % (TRASH // L)) * L)
                sidx_v[0, pl.ds(g * L, L)] = jnp.where(in_chunk, local, trash)
            pltpu.sync_copy(b_v, chunk_sh.at[sidx_v.at[0]], add=True)

        plsc.subcore_barrier()

        @pl.when(s < NS - 1)
        def _():
            pltpu.sync_copy(
                chunk_sh.at[pl.ds(s * A_PER_TILE, A_PER_TILE)],
                out_hbm.at[pl.ds(base + s * A_PER_TILE, A_PER_TILE)],
            )

        @pl.when((s == NS - 1) & (jnp.bool_(not is_final) | (c == 0)))
        def _():
            pltpu.sync_copy(
                chunk_sh.at[pl.ds((NS - 1) * A_PER_TILE, A_LAST)],
                out_hbm.at[pl.ds(base + (NS - 1) * A_PER_TILE, A_LAST)],
            )

        if is_final:
            @pl.when((s == NS - 1) & (c == 1))
            def _():
                pltpu.sync_copy(
                    chunk_sh.at[pl.ds((NS - 1) * A_PER_TILE, A_LAST_FINAL)],
                    out_hbm.at[pl.ds(base + (NS - 1) * A_PER_TILE, A_LAST_FINAL)],
                )

        if k != CHUNKS_PER_CORE - 1:
            plsc.subcore_barrier()


def kernel(index, A, B):
    return _scatter_add(index.astype(jnp.int32), A, B)


# speedup vs baseline: 1.1331x; 1.1331x over previous
import functools

import jax
import jax.numpy as jnp
from jax import lax
from jax.experimental import pallas as pl
from jax.experimental.pallas import tpu as pltpu
from jax.experimental.pallas import tpu_sc as plsc

N_ROWS = 100000
D = 64
B_ROWS = 16384

NC = 2
NS = 16
L = 16

CHUNKS_PER_CORE = 4
CHUNK = 12504
LAST_CHUNK = N_ROWS - (NC * CHUNKS_PER_CORE - 1) * CHUNK
TRASH = 64

B_SUB = 64
N_SUB = B_ROWS // (B_SUB * NS)
NBUF = 2

A_PER_TILE = 784
A_LAST = CHUNK - (NS - 1) * A_PER_TILE
A_LAST_FINAL = LAST_CHUNK - (NS - 1) * A_PER_TILE

_mesh = plsc.VectorSubcoreMesh(core_axis_name="c", subcore_axis_name="s")


@functools.partial(
    pl.kernel,
    mesh=_mesh,
    out_type=jax.ShapeDtypeStruct((N_ROWS, D), jnp.float32),
    scratch_types=[
        pltpu.VMEM((B_SUB, D), jnp.float32),
        pltpu.VMEM((B_SUB, D), jnp.float32),
        pltpu.VMEM((B_SUB,), jnp.int32),
        pltpu.VMEM((B_SUB,), jnp.int32),
        pltpu.VMEM((1, B_SUB), jnp.int32),
        pltpu.VMEM((1, B_SUB), jnp.int32),
        pltpu.VMEM_SHARED((CHUNK + TRASH, D), jnp.float32),
        pltpu.SemaphoreType.DMA,
        pltpu.SemaphoreType.DMA,
        pltpu.SemaphoreType.DMA,
        pltpu.SemaphoreType.DMA,
        pltpu.SemaphoreType.DMA,
    ],
    compiler_params=pltpu.CompilerParams(use_tc_tiling_on_sc=True),
)
def _scatter_add(idx_hbm, a_hbm, b_hbm, out_hbm,
                 b_v0, b_v1, i_v0, i_v1, sidx_v0, sidx_v1, chunk_sh,
                 sem_b0, sem_b1, sem_i0, sem_i1, sem_sc):
    c = lax.axis_index("c")
    s = lax.axis_index("s")
    lanes = lax.iota(jnp.int32, L)
    b_bank = (b_v0, b_v1)
    i_bank = (i_v0, i_v1)
    sidx_bank = (sidx_v0, sidx_v1)
    sem_b = (sem_b0, sem_b1)
    sem_i = (sem_i0, sem_i1)

    def stage_b(sub):
        p = sub % NBUF
        off = sub * (B_SUB * NS) + s * B_SUB
        return pltpu.make_async_copy(
            b_hbm.at[pl.ds(off, B_SUB)], b_bank[p], sem_b[p])

    def stage_i(sub):
        p = sub % NBUF
        off = sub * (B_SUB * NS) + s * B_SUB
        return pltpu.make_async_copy(
            idx_hbm.at[pl.ds(off, B_SUB)], i_bank[p], sem_i[p])

    for k in range(CHUNKS_PER_CORE):
        base = (c * CHUNKS_PER_CORE + k) * CHUNK
        is_final = k == CHUNKS_PER_CORE - 1

        stage_b(0).start()
        stage_i(0).start()
        stage_b(1).start()
        stage_i(1).start()

        @pl.when(s < NS - 1)
        def _():
            pltpu.sync_copy(
                a_hbm.at[pl.ds(base + s * A_PER_TILE, A_PER_TILE)],
                chunk_sh.at[pl.ds(s * A_PER_TILE, A_PER_TILE)],
            )

        @pl.when((s == NS - 1) & (jnp.bool_(not is_final) | (c == 0)))
        def _():
            pltpu.sync_copy(
                a_hbm.at[pl.ds(base + (NS - 1) * A_PER_TILE, A_LAST)],
                chunk_sh.at[pl.ds((NS - 1) * A_PER_TILE, A_LAST)],
            )

        if is_final:
            @pl.when((s == NS - 1) & (c == 1))
            def _():
                pltpu.sync_copy(
                    a_hbm.at[pl.ds(base + (NS - 1) * A_PER_TILE, A_LAST_FINAL)],
                    chunk_sh.at[pl.ds((NS - 1) * A_PER_TILE, A_LAST_FINAL)],
                )

        plsc.subcore_barrier()

        scatters = []
        for sub in range(N_SUB):
            p = sub % NBUF
            stage_b(sub).wait()
            stage_i(sub).wait()
            for g in range(B_SUB // L):
                v = i_bank[p][pl.ds(g * L, L)]
                local = v - base
                in_chunk = (local >= 0) & (local < CHUNK)
                trash = lanes + jnp.int32(CHUNK + (g % (TRASH // L)) * L)
                sidx_bank[p][0, pl.ds(g * L, L)] = (
                    jnp.where(in_chunk, local, trash))
            sc = pltpu.make_async_copy(
                b_bank[p], chunk_sh.at[sidx_bank[p].at[0]], sem_sc)
            sc.start(add=True)
            scatters.append(sc)
            if sub + NBUF < N_SUB:
                sc.wait()
                stage_b(sub + NBUF).start()
                stage_i(sub + NBUF).start()
        for sc in scatters[-NBUF:]:
            sc.wait()

        plsc.subcore_barrier()

        @pl.when(s < NS - 1)
        def _():
            pltpu.sync_copy(
                chunk_sh.at[pl.ds(s * A_PER_TILE, A_PER_TILE)],
                out_hbm.at[pl.ds(base + s * A_PER_TILE, A_PER_TILE)],
            )

        @pl.when((s == NS - 1) & (jnp.bool_(not is_final) | (c == 0)))
        def _():
            pltpu.sync_copy(
                chunk_sh.at[pl.ds((NS - 1) * A_PER_TILE, A_LAST)],
                out_hbm.at[pl.ds(base + (NS - 1) * A_PER_TILE, A_LAST)],
            )

        if is_final:
            @pl.when((s == NS - 1) & (c == 1))
            def _():
                pltpu.sync_copy(
                    chunk_sh.at[pl.ds((NS - 1) * A_PER_TILE, A_LAST_FINAL)],
                    out_hbm.at[pl.ds(base + (NS - 1) * A_PER_TILE, A_LAST_FINAL)],
                )

        if k != CHUNKS_PER_CORE - 1:
            plsc.subcore_barrier()


def kernel(index, A, B):
    return _scatter_add(index.astype(jnp.int32), A, B)


# speedup vs baseline: 1.1669x; 1.0298x over previous
import functools

import jax
import jax.numpy as jnp
from jax import lax
from jax.experimental import pallas as pl
from jax.experimental.pallas import tpu as pltpu
from jax.experimental.pallas import tpu_sc as plsc

N_ROWS = 100000
D = 64
B_ROWS = 16384

NC = 2
NS = 16
L = 16

CHUNKS_PER_CORE = 4
CHUNK = 12504
LAST_CHUNK = N_ROWS - (NC * CHUNKS_PER_CORE - 1) * CHUNK
TRASH = 64

B_SUB = 64
N_SUB = B_ROWS // (B_SUB * NS)
NBUF = 3

A_PER_TILE = 784
A_LAST = CHUNK - (NS - 1) * A_PER_TILE
A_LAST_FINAL = LAST_CHUNK - (NS - 1) * A_PER_TILE

_mesh = plsc.VectorSubcoreMesh(core_axis_name="c", subcore_axis_name="s")


@functools.partial(
    pl.kernel,
    mesh=_mesh,
    out_type=jax.ShapeDtypeStruct((N_ROWS, D), jnp.float32),
    scratch_types=[
        pltpu.VMEM((B_SUB, D), jnp.float32),
        pltpu.VMEM((B_SUB, D), jnp.float32),
        pltpu.VMEM((B_SUB, D), jnp.float32),
        pltpu.VMEM((B_SUB,), jnp.int32),
        pltpu.VMEM((B_SUB,), jnp.int32),
        pltpu.VMEM((B_SUB,), jnp.int32),
        pltpu.VMEM((1, B_SUB), jnp.int32),
        pltpu.VMEM((1, B_SUB), jnp.int32),
        pltpu.VMEM((1, B_SUB), jnp.int32),
        pltpu.VMEM_SHARED((CHUNK + TRASH, D), jnp.float32),
        pltpu.SemaphoreType.DMA,
        pltpu.SemaphoreType.DMA,
        pltpu.SemaphoreType.DMA,
        pltpu.SemaphoreType.DMA,
        pltpu.SemaphoreType.DMA,
        pltpu.SemaphoreType.DMA,
        pltpu.SemaphoreType.DMA,
    ],
    compiler_params=pltpu.CompilerParams(use_tc_tiling_on_sc=True),
)
def _scatter_add(idx_hbm, a_hbm, b_hbm, out_hbm,
                 b_v0, b_v1, b_v2, i_v0, i_v1, i_v2,
                 sidx_v0, sidx_v1, sidx_v2, chunk_sh,
                 sem_b0, sem_b1, sem_b2, sem_i0, sem_i1, sem_i2, sem_sc):
    c = lax.axis_index("c")
    s = lax.axis_index("s")
    lanes = lax.iota(jnp.int32, L)
    b_bank = (b_v0, b_v1, b_v2)
    i_bank = (i_v0, i_v1, i_v2)
    sidx_bank = (sidx_v0, sidx_v1, sidx_v2)
    sem_b = (sem_b0, sem_b1, sem_b2)
    sem_i = (sem_i0, sem_i1, sem_i2)

    def stage_b(sub):
        p = sub % NBUF
        off = sub * (B_SUB * NS) + s * B_SUB
        return pltpu.make_async_copy(
            b_hbm.at[pl.ds(off, B_SUB)], b_bank[p], sem_b[p])

    def stage_i(sub):
        p = sub % NBUF
        off = sub * (B_SUB * NS) + s * B_SUB
        return pltpu.make_async_copy(
            idx_hbm.at[pl.ds(off, B_SUB)], i_bank[p], sem_i[p])

    for k in range(CHUNKS_PER_CORE):
        base = (c * CHUNKS_PER_CORE + k) * CHUNK
        is_final = k == CHUNKS_PER_CORE - 1

        for pre in range(NBUF):
            stage_b(pre).start()
            stage_i(pre).start()

        @pl.when(s < NS - 1)
        def _():
            pltpu.sync_copy(
                a_hbm.at[pl.ds(base + s * A_PER_TILE, A_PER_TILE)],
                chunk_sh.at[pl.ds(s * A_PER_TILE, A_PER_TILE)],
            )

        @pl.when((s == NS - 1) & (jnp.bool_(not is_final) | (c == 0)))
        def _():
            pltpu.sync_copy(
                a_hbm.at[pl.ds(base + (NS - 1) * A_PER_TILE, A_LAST)],
                chunk_sh.at[pl.ds((NS - 1) * A_PER_TILE, A_LAST)],
            )

        if is_final:
            @pl.when((s == NS - 1) & (c == 1))
            def _():
                pltpu.sync_copy(
                    a_hbm.at[pl.ds(base + (NS - 1) * A_PER_TILE, A_LAST_FINAL)],
                    chunk_sh.at[pl.ds((NS - 1) * A_PER_TILE, A_LAST_FINAL)],
                )

        plsc.subcore_barrier()

        scatters = []
        for sub in range(N_SUB):
            p = sub % NBUF
            stage_b(sub).wait()
            stage_i(sub).wait()
            for g in range(B_SUB // L):
                v = i_bank[p][pl.ds(g * L, L)]
                local = v - base
                in_chunk = (local >= 0) & (local < CHUNK)
                trash = lanes + jnp.int32(CHUNK + (g % (TRASH // L)) * L)
                sidx_bank[p][0, pl.ds(g * L, L)] = (
                    jnp.where(in_chunk, local, trash))
            sc = pltpu.make_async_copy(
                b_bank[p], chunk_sh.at[sidx_bank[p].at[0]], sem_sc)
            sc.start(add=True)
            scatters.append(sc)
            if sub >= 1 and sub - 1 + NBUF < N_SUB:
                scatters[sub - 1].wait()
                stage_b(sub - 1 + NBUF).start()
                stage_i(sub - 1 + NBUF).start()
        for sc in scatters[-(NBUF - 1):] + [scatters[N_SUB - NBUF]]:
            sc.wait()

        plsc.subcore_barrier()

        @pl.when(s < NS - 1)
        def _():
            pltpu.sync_copy(
                chunk_sh.at[pl.ds(s * A_PER_TILE, A_PER_TILE)],
                out_hbm.at[pl.ds(base + s * A_PER_TILE, A_PER_TILE)],
            )

        @pl.when((s == NS - 1) & (jnp.bool_(not is_final) | (c == 0)))
        def _():
            pltpu.sync_copy(
                chunk_sh.at[pl.ds((NS - 1) * A_PER_TILE, A_LAST)],
                out_hbm.at[pl.ds(base + (NS - 1) * A_PER_TILE, A_LAST)],
            )

        if is_final:
            @pl.when((s == NS - 1) & (c == 1))
            def _():
                pltpu.sync_copy(
                    chunk_sh.at[pl.ds((NS - 1) * A_PER_TILE, A_LAST_FINAL)],
                    out_hbm.at[pl.ds(base + (NS - 1) * A_PER_TILE, A_LAST_FINAL)],
                )

        if k != CHUNKS_PER_CORE - 1:
            plsc.subcore_barrier()


def kernel(index, A, B):
    return _scatter_add(index.astype(jnp.int32), A, B)
